# baseline (speedup 1.0000x reference)
import jax
import jax.numpy as jnp
from jax import lax
from jax.experimental import pallas as pl
from jax.experimental.pallas import tpu as pltpu
from jax.experimental.pallas import tpu_sc as plsc

_N = 10000
_E = 160000
_G = 64

_NC = 2
_NS = 16
_NW = _NC * _NS
_CH = 128
_K0 = 62
_K1 = 18
_KMAX = max(_K0, _K1)
_E0 = _K0 * _NS * _CH
_ACC_ROWS = 10112
_ZROWS = _ACC_ROWS // _NS

_RB = 400
_NB = _N // _RB


def _sc_mesh():
    return plsc.VectorSubcoreMesh(core_axis_name="c", subcore_axis_name="s",
                                  num_cores=_NC, num_subcores=_NS)


def _deg_partials(dst_w):
    zeros = jnp.zeros((_ZROWS, 128), jnp.float32)
    ones = jnp.ones((_CH, 128), jnp.float32)

    def body(dstw, ones_hbm, zeros_hbm, out, dall, ones_v, acc):
        c = lax.axis_index("c")
        s = lax.axis_index("s")
        kc = jnp.where(c == 0, _K0, _K1)
        pltpu.sync_copy(zeros_hbm, acc.at[pl.ds(s * _ZROWS, _ZROWS)])
        pltpu.sync_copy(ones_hbm, ones_v)
        pltpu.sync_copy(dstw.at[c, s], dall)
        plsc.subcore_barrier()

        def step(j, carry):
            pltpu.sync_copy(ones_v, acc.at[dall.at[j]], add=True)
            return carry

        lax.fori_loop(0, kc, step, 0)
        plsc.subcore_barrier()
        pltpu.sync_copy(acc.at[pl.ds(s * _ZROWS, _ZROWS)],
                        out.at[c, pl.ds(s * _ZROWS, _ZROWS)])

    f = pl.kernel(
        body,
        out_type=jax.ShapeDtypeStruct((_NC, _ACC_ROWS, 128), jnp.float32),
        mesh=_sc_mesh(),
        scratch_types=[pltpu.VMEM((_KMAX, _CH), jnp.int32),
                       pltpu.VMEM((_CH, 128), jnp.float32),
                       pltpu.VMEM_SHARED((_ACC_ROWS, 128), jnp.float32)])
    return f(dst_w, ones, zeros)


def _agg_partials(src_w, dst_w, tables):
    ncb = len(tables)
    dcol = tables[0].shape[1]
    zeros = jnp.zeros((_ZROWS, dcol), jnp.float32)

    nbuf = 2

    def body(*refs):
        srcw, dstw = refs[0], refs[1]
        tabs = refs[2:2 + ncb]
        zeros_hbm = refs[2 + ncb]
        outs = refs[3 + ncb:3 + 2 * ncb]
        rest = refs[3 + 2 * ncb:]
        sall, dall = rest[0], rest[1]
        rows = rest[2:2 + nbuf]
        acc = rest[2 + nbuf]
        gsems = rest[3 + nbuf:3 + 2 * nbuf]
        ssems = rest[3 + 2 * nbuf:3 + 3 * nbuf]
        c = lax.axis_index("c")
        s = lax.axis_index("s")
        kc2 = jnp.where(c == 0, _K0 // nbuf, _K1 // nbuf)
        pltpu.sync_copy(srcw.at[c, s], sall)
        pltpu.sync_copy(dstw.at[c, s], dall)
        for cb in range(ncb):
            pltpu.sync_copy(zeros_hbm, acc.at[pl.ds(s * _ZROWS, _ZROWS)])
            plsc.subcore_barrier()
            tab = tabs[cb]
            for b in range(nbuf):
                pltpu.async_copy(tab.at[sall.at[b]], rows[b], gsems[b])

            def step(t, carry, tab=tab, kc2=kc2):
                descs = []
                for b in range(nbuf):
                    pltpu.make_async_copy(tab.at[sall.at[b]], rows[b],
                                          gsems[b]).wait()
                    descs.append(pltpu.async_copy(
                        rows[b], acc.at[dall.at[t * nbuf + b]], ssems[b],
                        add=True))
                for b in range(nbuf):
                    descs[b].wait()

                    @pl.when(t < kc2 - 1)
                    def _(b=b, t=t):
                        pltpu.async_copy(
                            tab.at[sall.at[t * nbuf + nbuf + b]],
                            rows[b], gsems[b])
                return carry

            lax.fori_loop(0, kc2, step, 0)
            plsc.subcore_barrier()
            pltpu.sync_copy(acc.at[pl.ds(s * _ZROWS, _ZROWS)],
                            outs[cb].at[c, pl.ds(s * _ZROWS, _ZROWS)])
            plsc.subcore_barrier()

    f = pl.kernel(
        body,
        out_type=[jax.ShapeDtypeStruct((_NC, _ACC_ROWS, dcol),
                                       jnp.float32)] * ncb,
        mesh=_sc_mesh(),
        scratch_types=[pltpu.VMEM((_KMAX, _CH), jnp.int32),
                       pltpu.VMEM((_KMAX, _CH), jnp.int32)] +
                      [pltpu.VMEM((_CH, dcol), jnp.float32)] * nbuf +
                      [pltpu.VMEM_SHARED((_ACC_ROWS, dcol), jnp.float32)] +
                      [pltpu.SemaphoreType.DMA] * (2 * nbuf))
    out = f(src_w, dst_w, *tables, zeros)
    return list(out) if isinstance(out, (list, tuple)) else [out]


def _stage_a_body(x_ref, w_ref, dp_ref, dinv_ref, h0, h1, h2, h3):
    deg = 1.0 + dp_ref[0, :, 0:1] + dp_ref[1, :, 0:1]
    dinv = lax.rsqrt(deg)
    h = jnp.dot(x_ref[...], w_ref[...],
                preferred_element_type=jnp.float32) * dinv
    dinv_ref[...] = dinv
    for cb, hr in enumerate((h0, h1, h2, h3)):
        hr[...] = h[:, cb * 128:(cb + 1) * 128]


def _stage_a(x, w1, degp):
    din = x.shape[1]
    return pl.pallas_call(
        _stage_a_body,
        grid=(_NB,),
        in_specs=[pl.BlockSpec((_RB, din), lambda i: (i, 0)),
                  pl.BlockSpec((din, 512), lambda i: (0, 0)),
                  pl.BlockSpec((2, _RB, 128), lambda i: (0, i, 0))],
        out_specs=[pl.BlockSpec((_RB, 1), lambda i: (i, 0))] +
                  [pl.BlockSpec((_RB, 128), lambda i: (i, 0))] * 4,
        out_shape=[jax.ShapeDtypeStruct((_N, 1), jnp.float32)] +
                  [jax.ShapeDtypeStruct((_N, 128), jnp.float32)] * 4,
    )(x, w1, degp)


def _stats(ps, hs, dinv, b2d):
    ncb = len(hs)
    dcol = hs[0].shape[1]

    def body(*refs):
        p_refs = refs[0:ncb]
        h_refs = refs[ncb:2 * ncb]
        dinv_ref = refs[2 * ncb]
        b_ref = refs[2 * ncb + 1]
        zouts = refs[2 * ncb + 2:2 * ncb + 2 + ncb]
        st = refs[-1]
        i = pl.program_id(0)

        @pl.when(i == 0)
        def _():
            st[...] = jnp.zeros_like(st)

        dinv = dinv_ref[...]
        for cb in range(ncb):
            z = dinv * (p_refs[cb][0] + p_refs[cb][1] + h_refs[cb][...]) \
                + b_ref[cb]
            zouts[cb][...] = z
            sl = pl.ds(cb * dcol, dcol)
            st[0:1, sl] = st[0:1, sl] + jnp.sum(z, 0, keepdims=True)
            st[1:2, sl] = st[1:2, sl] + jnp.sum(z * z, 0, keepdims=True)

    outs = pl.pallas_call(
        body,
        grid=(_NB,),
        in_specs=[pl.BlockSpec((2, _RB, dcol), lambda i: (0, i, 0))] * ncb +
                 [pl.BlockSpec((_RB, dcol), lambda i: (i, 0))] * ncb +
                 [pl.BlockSpec((_RB, 1), lambda i: (i, 0)),
                  pl.BlockSpec((ncb, dcol), lambda i: (0, 0))],
        out_specs=[pl.BlockSpec((_RB, dcol), lambda i: (i, 0))] * ncb +
                  [pl.BlockSpec((2, ncb * dcol), lambda i: (0, 0))],
        out_shape=[jax.ShapeDtypeStruct((_N, dcol), jnp.float32)] * ncb +
                  [jax.ShapeDtypeStruct((2, ncb * dcol), jnp.float32)],
    )(*ps, *hs, dinv, b2d)
    return outs[:ncb], outs[ncb]


def _transform(zs, st, g, be, w, dinv, ncb_out, dcol_out):
    ncb_in = len(zs)
    dcol_in = zs[0].shape[1]
    din = ncb_in * dcol_in
    dn = w.shape[1]

    def body(*refs):
        z_refs = refs[0:ncb_in]
        st_ref = refs[ncb_in]
        g_ref = refs[ncb_in + 1]
        be_ref = refs[ncb_in + 2]
        w_ref = refs[ncb_in + 3]
        dinv_ref = refs[ncb_in + 4]
        outs = refs[ncb_in + 5:]
        mu = st_ref[0:1, :] * (1.0 / _N)
        var = st_ref[1:2, :] * (1.0 / _N) - mu * mu
        scale = g_ref[...] * lax.rsqrt(var + 1e-5)
        if ncb_in > 1:
            z = jnp.concatenate([zr[...] for zr in z_refs], axis=1)
        else:
            z = z_refs[0][...]
        a = jnp.maximum((z - mu) * scale + be_ref[...], 0.0)
        h = jnp.dot(a, w_ref[...],
                    preferred_element_type=jnp.float32) * dinv_ref[...]
        pad = ncb_out * dcol_out - dn
        if pad:
            h = jnp.concatenate(
                [h, jnp.zeros((_RB, pad), jnp.float32)], axis=1)
        for cb, o in enumerate(outs):
            o[...] = h[:, cb * dcol_out:(cb + 1) * dcol_out]

    return pl.pallas_call(
        body,
        grid=(_NB,),
        in_specs=[pl.BlockSpec((_RB, dcol_in), lambda i: (i, 0))] * ncb_in +
                 [pl.BlockSpec((2, din), lambda i: (0, 0)),
                  pl.BlockSpec((1, din), lambda i: (0, 0)),
                  pl.BlockSpec((1, din), lambda i: (0, 0)),
                  pl.BlockSpec((din, dn), lambda i: (0, 0)),
                  pl.BlockSpec((_RB, 1), lambda i: (i, 0))],
        out_specs=[pl.BlockSpec((_RB, dcol_out), lambda i: (i, 0))] * ncb_out,
        out_shape=[jax.ShapeDtypeStruct((_N, dcol_out), jnp.float32)] * ncb_out,
    )(*zs, st, g, be, w, dinv)


def _final(p3, h3, dinv, b, g, be, batch3d):
    def body(p_ref, h_ref, dinv_ref, b_ref, g_ref, be_ref, bt_ref,
             out_ref, acc, st):
        i = pl.program_id(0)

        @pl.when(i == 0)
        def _():
            acc[...] = jnp.zeros_like(acc)
            st[...] = jnp.zeros_like(st)

        z = dinv_ref[...] * (p_ref[0] + p_ref[1] + h_ref[...]) + b_ref[...]
        st[0:1, :] = st[0:1, :] + jnp.sum(z, 0, keepdims=True)
        st[1:2, :] = st[1:2, :] + jnp.sum(z * z, 0, keepdims=True)
        bt = jnp.broadcast_to(bt_ref[0], (_G, _RB))
        oh_t = (lax.broadcasted_iota(jnp.int32, (_G, _RB), 0) == bt
                ).astype(jnp.float32)
        zaug = jnp.concatenate(
            [z[:, 0:64], jnp.ones((_RB, 64), jnp.float32)], axis=1)
        acc[...] = acc[...] + jnp.dot(oh_t, zaug,
                                      preferred_element_type=jnp.float32)

        @pl.when(i == _NB - 1)
        def _():
            mu = st[0:1, 0:64] * (1.0 / _N)
            var = st[1:2, 0:64] * (1.0 / _N) - mu * mu
            scale = g_ref[0:1, 0:64] * lax.rsqrt(var + 1e-5)
            s = acc[...]
            cnt = jnp.maximum(s[:, 64:65], 1.0)
            pooled = (s[:, 0:64] / cnt - mu) * scale + be_ref[0:1, 0:64]
            m = jnp.max(pooled, axis=1, keepdims=True)
            e = jnp.exp(pooled - m)
            out_ref[...] = (pooled - m) - jnp.log(
                jnp.sum(e, axis=1, keepdims=True))

    return pl.pallas_call(
        body,
        grid=(_NB,),
        in_specs=[pl.BlockSpec((2, _RB, 128), lambda i: (0, i, 0)),
                  pl.BlockSpec((_RB, 128), lambda i: (i, 0)),
                  pl.BlockSpec((_RB, 1), lambda i: (i, 0)),
                  pl.BlockSpec((1, 128), lambda i: (0, 0)),
                  pl.BlockSpec((1, 128), lambda i: (0, 0)),
                  pl.BlockSpec((1, 128), lambda i: (0, 0)),
                  pl.BlockSpec((1, 1, _RB), lambda i: (i, 0, 0))],
        out_specs=pl.BlockSpec((_G, 64), lambda i: (0, 0)),
        out_shape=jax.ShapeDtypeStruct((_G, 64), jnp.float32),
        scratch_shapes=[pltpu.VMEM((_G, 128), jnp.float32),
                        pltpu.VMEM((2, 128), jnp.float32)],
    )(p3, h3, dinv, b, g, be, batch3d)


def kernel(x, edge_index, batch, W1, b1, g1, be1, W2, b2, g2, be2,
           W3, b3, g3, be3):
    src = edge_index[0]
    dst = edge_index[1]

    def _split(a, fill):
        parts = []
        for lo, hi, k in ((0, _E0, _K0), (_E0, _E, _K1)):
            seg = a[lo:hi]
            pad = k * _NS * _CH - (hi - lo)
            seg = jnp.concatenate(
                [seg, jnp.full((pad,), fill, jnp.int32)]).reshape(
                    _NS, k, _CH)
            seg = jnp.pad(seg, ((0, 0), (0, _KMAX - k), (0, 0)),
                          constant_values=fill)
            parts.append(seg)
        return jnp.stack(parts)

    src_w = _split(src, 0)
    dst_w = _split(dst, _N)

    degp = _deg_partials(dst_w)
    dinv, h0, h1, h2, h3 = _stage_a(x, W1, degp)
    hs1 = (h0, h1, h2, h3)

    p1 = _agg_partials(src_w, dst_w, hs1)
    z1, st1 = _stats(p1, hs1, dinv, b1.reshape(4, 128))
    hs2 = _transform(z1, st1, g1.reshape(1, 512), be1.reshape(1, 512),
                     W2, dinv, ncb_out=4, dcol_out=128)

    p2 = _agg_partials(src_w, dst_w, hs2)
    z2, st2 = _stats(p2, hs2, dinv, b2.reshape(4, 128))
    hs3 = _transform(z2, st2, g2.reshape(1, 512), be2.reshape(1, 512),
                     W3, dinv, ncb_out=1, dcol_out=128)
    zpad = jnp.zeros((64,), jnp.float32)
    b3p = jnp.concatenate([b3, zpad]).reshape(1, 128)
    g3p = jnp.concatenate([g3, jnp.ones((64,), jnp.float32)]).reshape(1, 128)
    be3p = jnp.concatenate([be3, zpad]).reshape(1, 128)

    p3 = _agg_partials(src_w, dst_w, tuple(hs3))
    return _final(p3[0], hs3[0], dinv, b3p, g3p, be3p,
                  batch.reshape(_NB, 1, _RB))


# speedup vs baseline: 1.0106x; 1.0106x over previous
import jax
import jax.numpy as jnp
from jax import lax
from jax.experimental import pallas as pl
from jax.experimental.pallas import tpu as pltpu
from jax.experimental.pallas import tpu_sc as plsc

_N = 10000
_E = 160000
_G = 64

_NC = 2
_NS = 16
_NW = _NC * _NS
_CH = 128
_K0 = 62
_K1 = 18
_KMAX = max(_K0, _K1)
_E0 = _K0 * _NS * _CH
_ACC_ROWS = 10112
_ZROWS = _ACC_ROWS // _NS

_RB = 400
_NB = _N // _RB


def _sc_mesh():
    return plsc.VectorSubcoreMesh(core_axis_name="c", subcore_axis_name="s",
                                  num_cores=_NC, num_subcores=_NS)


def _deg_partials(dst_w):
    zeros = jnp.zeros((_ZROWS, 128), jnp.float32)
    ones = jnp.ones((_CH, 128), jnp.float32)

    def body(dstw, ones_hbm, zeros_hbm, out, dall, ones_v, acc):
        c = lax.axis_index("c")
        s = lax.axis_index("s")
        kc = jnp.where(c == 0, _K0, _K1)
        pltpu.sync_copy(zeros_hbm, acc.at[pl.ds(s * _ZROWS, _ZROWS)])
        pltpu.sync_copy(ones_hbm, ones_v)
        pltpu.sync_copy(dstw.at[c, s], dall)
        plsc.subcore_barrier()

        def step(j, carry):
            pltpu.sync_copy(ones_v, acc.at[dall.at[j]], add=True)
            return carry

        lax.fori_loop(0, kc, step, 0)
        plsc.subcore_barrier()
        pltpu.sync_copy(acc.at[pl.ds(s * _ZROWS, _ZROWS)],
                        out.at[c, pl.ds(s * _ZROWS, _ZROWS)])

    f = pl.kernel(
        body,
        out_type=jax.ShapeDtypeStruct((_NC, _ACC_ROWS, 128), jnp.float32),
        mesh=_sc_mesh(),
        scratch_types=[pltpu.VMEM((_KMAX, _CH), jnp.int32),
                       pltpu.VMEM((_CH, 128), jnp.float32),
                       pltpu.VMEM_SHARED((_ACC_ROWS, 128), jnp.float32)])
    return f(dst_w, ones, zeros)


def _agg_partials(src_w, dst_w, tables):
    ncb = len(tables)
    dcol = tables[0].shape[1]
    zeros = jnp.zeros((_ZROWS, dcol), jnp.float32)

    nbuf = 2

    def body(*refs):
        srcw, dstw = refs[0], refs[1]
        tabs = refs[2:2 + ncb]
        zeros_hbm = refs[2 + ncb]
        outs = refs[3 + ncb:3 + 2 * ncb]
        rest = refs[3 + 2 * ncb:]
        sall, dall = rest[0], rest[1]
        rows = rest[2:2 + nbuf]
        acc = rest[2 + nbuf]
        gsems = rest[3 + nbuf:3 + 2 * nbuf]
        ssems = rest[3 + 2 * nbuf:3 + 3 * nbuf]
        c = lax.axis_index("c")
        s = lax.axis_index("s")
        kc2 = jnp.where(c == 0, _K0 // nbuf, _K1 // nbuf)
        pltpu.sync_copy(srcw.at[c, s], sall)
        pltpu.sync_copy(dstw.at[c, s], dall)
        for cb in range(ncb):
            pltpu.sync_copy(zeros_hbm, acc.at[pl.ds(s * _ZROWS, _ZROWS)])
            plsc.subcore_barrier()
            tab = tabs[cb]
            for b in range(nbuf):
                pltpu.async_copy(tab.at[sall.at[b]], rows[b], gsems[b])

            def step(t, carry, tab=tab, kc2=kc2):
                descs = []
                for b in range(nbuf):
                    pltpu.make_async_copy(tab.at[sall.at[b]], rows[b],
                                          gsems[b]).wait()
                    descs.append(pltpu.async_copy(
                        rows[b], acc.at[dall.at[t * nbuf + b]], ssems[b],
                        add=True))
                for b in range(nbuf):
                    descs[b].wait()

                    @pl.when(t < kc2 - 1)
                    def _(b=b, t=t):
                        pltpu.async_copy(
                            tab.at[sall.at[t * nbuf + nbuf + b]],
                            rows[b], gsems[b])
                return carry

            lax.fori_loop(0, kc2, step, 0)
            plsc.subcore_barrier()
            pltpu.sync_copy(acc.at[pl.ds(s * _ZROWS, _ZROWS)],
                            outs[cb].at[c, pl.ds(s * _ZROWS, _ZROWS)])
            plsc.subcore_barrier()

    f = pl.kernel(
        body,
        out_type=[jax.ShapeDtypeStruct((_NC, _ACC_ROWS, dcol),
                                       jnp.float32)] * ncb,
        mesh=_sc_mesh(),
        scratch_types=[pltpu.VMEM((_KMAX, _CH), jnp.int32),
                       pltpu.VMEM((_KMAX, _CH), jnp.int32)] +
                      [pltpu.VMEM((_CH, dcol), jnp.float32)] * nbuf +
                      [pltpu.VMEM_SHARED((_ACC_ROWS, dcol), jnp.float32)] +
                      [pltpu.SemaphoreType.DMA] * (2 * nbuf))
    out = f(src_w, dst_w, *tables, zeros)
    return list(out) if isinstance(out, (list, tuple)) else [out]


def _stage_a_body(x_ref, w_ref, dp_ref, dinv_ref, h0, h1, h2, h3):
    deg = 1.0 + dp_ref[0, :, 0:1] + dp_ref[1, :, 0:1]
    dinv = lax.rsqrt(deg)
    h = jnp.dot(x_ref[...], w_ref[...],
                preferred_element_type=jnp.float32) * dinv
    dinv_ref[...] = dinv
    for cb, hr in enumerate((h0, h1, h2, h3)):
        hr[...] = h[:, cb * 128:(cb + 1) * 128]


def _stage_a(x, w1, degp):
    din = x.shape[1]
    return pl.pallas_call(
        _stage_a_body,
        grid=(_NB,),
        in_specs=[pl.BlockSpec((_RB, din), lambda i: (i, 0)),
                  pl.BlockSpec((din, 512), lambda i: (0, 0)),
                  pl.BlockSpec((2, _RB, 128), lambda i: (0, i, 0))],
        out_specs=[pl.BlockSpec((_RB, 1), lambda i: (i, 0))] +
                  [pl.BlockSpec((_RB, 128), lambda i: (i, 0))] * 4,
        out_shape=[jax.ShapeDtypeStruct((_N, 1), jnp.float32)] +
                  [jax.ShapeDtypeStruct((_N, 128), jnp.float32)] * 4,
    )(x, w1, degp)


def _stats(ps, hs, dinv, b2d):
    ncb = len(hs)
    dcol = hs[0].shape[1]

    def body(*refs):
        p_refs = refs[0:ncb]
        h_refs = refs[ncb:2 * ncb]
        dinv_ref = refs[2 * ncb]
        b_ref = refs[2 * ncb + 1]
        zouts = refs[2 * ncb + 2:2 * ncb + 2 + ncb]
        st = refs[-1]
        i = pl.program_id(0)

        @pl.when(i == 0)
        def _():
            st[...] = jnp.zeros_like(st)

        dinv = dinv_ref[...]
        for cb in range(ncb):
            z = dinv * (p_refs[cb][0] + p_refs[cb][1] + h_refs[cb][...]) \
                + b_ref[cb]
            zouts[cb][...] = z
            sl = pl.ds(cb * dcol, dcol)
            st[0:1, sl] = st[0:1, sl] + jnp.sum(z, 0, keepdims=True)
            st[1:2, sl] = st[1:2, sl] + jnp.sum(z * z, 0, keepdims=True)

    outs = pl.pallas_call(
        body,
        grid=(_NB,),
        in_specs=[pl.BlockSpec((2, _RB, dcol), lambda i: (0, i, 0))] * ncb +
                 [pl.BlockSpec((_RB, dcol), lambda i: (i, 0))] * ncb +
                 [pl.BlockSpec((_RB, 1), lambda i: (i, 0)),
                  pl.BlockSpec((ncb, dcol), lambda i: (0, 0))],
        out_specs=[pl.BlockSpec((_RB, dcol), lambda i: (i, 0))] * ncb +
                  [pl.BlockSpec((2, ncb * dcol), lambda i: (0, 0))],
        out_shape=[jax.ShapeDtypeStruct((_N, dcol), jnp.float32)] * ncb +
                  [jax.ShapeDtypeStruct((2, ncb * dcol), jnp.float32)],
    )(*ps, *hs, dinv, b2d)
    return outs[:ncb], outs[ncb]


def _transform(zs, st, g, be, w, dinv, ncb_out, dcol_out):
    ncb_in = len(zs)
    dcol_in = zs[0].shape[1]
    din = ncb_in * dcol_in
    dn = w.shape[1]

    def body(*refs):
        z_refs = refs[0:ncb_in]
        st_ref = refs[ncb_in]
        g_ref = refs[ncb_in + 1]
        be_ref = refs[ncb_in + 2]
        w_ref = refs[ncb_in + 3]
        dinv_ref = refs[ncb_in + 4]
        outs = refs[ncb_in + 5:]
        mu = st_ref[0:1, :] * (1.0 / _N)
        var = st_ref[1:2, :] * (1.0 / _N) - mu * mu
        scale = g_ref[...] * lax.rsqrt(var + 1e-5)
        if ncb_in > 1:
            z = jnp.concatenate([zr[...] for zr in z_refs], axis=1)
        else:
            z = z_refs[0][...]
        a = jnp.maximum((z - mu) * scale + be_ref[...], 0.0)
        h = jnp.dot(a, w_ref[...],
                    preferred_element_type=jnp.float32) * dinv_ref[...]
        pad = ncb_out * dcol_out - dn
        if pad:
            h = jnp.concatenate(
                [h, jnp.zeros((_RB, pad), jnp.float32)], axis=1)
        for cb, o in enumerate(outs):
            o[...] = h[:, cb * dcol_out:(cb + 1) * dcol_out]

    return pl.pallas_call(
        body,
        grid=(_NB,),
        in_specs=[pl.BlockSpec((_RB, dcol_in), lambda i: (i, 0))] * ncb_in +
                 [pl.BlockSpec((2, din), lambda i: (0, 0)),
                  pl.BlockSpec((1, din), lambda i: (0, 0)),
                  pl.BlockSpec((1, din), lambda i: (0, 0)),
                  pl.BlockSpec((din, dn), lambda i: (0, 0)),
                  pl.BlockSpec((_RB, 1), lambda i: (i, 0))],
        out_specs=[pl.BlockSpec((_RB, dcol_out), lambda i: (i, 0))] * ncb_out,
        out_shape=[jax.ShapeDtypeStruct((_N, dcol_out), jnp.float32)] * ncb_out,
    )(*zs, st, g, be, w, dinv)


def _final(z3, st3, g, be, batch3d):
    def body(z_ref, st_ref, g_ref, be_ref, b_ref, out_ref, acc):
        i = pl.program_id(0)

        @pl.when(i == 0)
        def _():
            acc[...] = jnp.zeros_like(acc)

        mu = st_ref[0:1, :] * (1.0 / _N)
        var = st_ref[1:2, :] * (1.0 / _N) - mu * mu
        scale = g_ref[...] * lax.rsqrt(var + 1e-5)
        z = ((z_ref[...] - mu) * scale + be_ref[...])[:, 0:64]
        bt = jnp.broadcast_to(b_ref[0], (_G, _RB))
        oh_t = (lax.broadcasted_iota(jnp.int32, (_G, _RB), 0) == bt
                ).astype(jnp.float32)
        zaug = jnp.concatenate([z, jnp.ones((_RB, 64), jnp.float32)], axis=1)
        acc[...] = acc[...] + jnp.dot(oh_t, zaug,
                                      preferred_element_type=jnp.float32)

        @pl.when(i == _NB - 1)
        def _():
            s = acc[...]
            cnt = jnp.maximum(s[:, 64:65], 1.0)
            pooled = s[:, 0:64] / cnt
            m = jnp.max(pooled, axis=1, keepdims=True)
            e = jnp.exp(pooled - m)
            out_ref[...] = (pooled - m) - jnp.log(
                jnp.sum(e, axis=1, keepdims=True))

    return pl.pallas_call(
        body,
        grid=(_NB,),
        in_specs=[pl.BlockSpec((_RB, 128), lambda i: (i, 0)),
                  pl.BlockSpec((2, 128), lambda i: (0, 0)),
                  pl.BlockSpec((1, 128), lambda i: (0, 0)),
                  pl.BlockSpec((1, 128), lambda i: (0, 0)),
                  pl.BlockSpec((1, 1, _RB), lambda i: (i, 0, 0))],
        out_specs=pl.BlockSpec((_G, 64), lambda i: (0, 0)),
        out_shape=jax.ShapeDtypeStruct((_G, 64), jnp.float32),
        scratch_shapes=[pltpu.VMEM((_G, 128), jnp.float32)],
    )(z3, st3, g, be, batch3d)


def kernel(x, edge_index, batch, W1, b1, g1, be1, W2, b2, g2, be2,
           W3, b3, g3, be3):
    src = edge_index[0]
    dst = edge_index[1]

    def _split(a, fill):
        parts = []
        for lo, hi, k in ((0, _E0, _K0), (_E0, _E, _K1)):
            seg = a[lo:hi]
            pad = k * _NS * _CH - (hi - lo)
            seg = jnp.concatenate(
                [seg, jnp.full((pad,), fill, jnp.int32)]).reshape(
                    _NS, k, _CH)
            seg = jnp.pad(seg, ((0, 0), (0, _KMAX - k), (0, 0)),
                          constant_values=fill)
            parts.append(seg)
        return jnp.stack(parts)

    src_w = _split(src, 0)
    dst_w = _split(dst, _N)

    degp = _deg_partials(dst_w)
    dinv, h0, h1, h2, h3 = _stage_a(x, W1, degp)
    hs1 = (h0, h1, h2, h3)

    p1 = _agg_partials(src_w, dst_w, hs1)
    z1, st1 = _stats(p1, hs1, dinv, b1.reshape(4, 128))
    hs2 = _transform(z1, st1, g1.reshape(1, 512), be1.reshape(1, 512),
                     W2, dinv, ncb_out=4, dcol_out=128)

    p2 = _agg_partials(src_w, dst_w, hs2)
    z2, st2 = _stats(p2, hs2, dinv, b2.reshape(4, 128))
    hs3 = _transform(z2, st2, g2.reshape(1, 512), be2.reshape(1, 512),
                     W3, dinv, ncb_out=1, dcol_out=128)
    zpad = jnp.zeros((64,), jnp.float32)
    b3p = jnp.concatenate([b3, zpad]).reshape(1, 128)
    g3p = jnp.concatenate([g3, jnp.ones((64,), jnp.float32)]).reshape(1, 128)
    be3p = jnp.concatenate([be3, zpad]).reshape(1, 128)

    p3 = _agg_partials(src_w, dst_w, tuple(hs3))
    z3, st3 = _stats(p3, tuple(hs3), dinv, b3p)
    return _final(z3[0], st3, g3p, be3p, batch.reshape(_NB, 1, _RB))
